# baseline (speedup 1.0000x reference)
import jax
import jax.numpy as jnp
import numpy as np
from jax.experimental import pallas as pl
from jax.experimental.pallas import tpu as pltpu

_SINKHORN_ITERS = 30
_I0 = np.int32(0)
_EPS = 1e-8
_TILE = 2048
_CHUNK = 512


def _router_kernel(hs_ref, W_ref, b_ref, logits_ref, aff_ref, idx_ref, C_scr):
    i = pl.program_id(0)
    n = pl.num_programs(0)
    tile = hs_ref.shape[0]

    logits = (
        jnp.dot(hs_ref[...], W_ref[...], preferred_element_type=jnp.float32)
        + b_ref[...]
    )
    logits_ref[...] = logits
    aff_ref[...] = jax.nn.sigmoid(logits)
    C_scr[pl.ds(i * tile, tile), :] = jnp.exp(logits)

    @pl.when(i == n - 1)
    def _finalize():
        T, E = C_scr.shape
        n_chunks = T // _CHUNK

        def iter_body(_, d1):
            def chunk_body(j, colsum):
                Cc = C_scr[pl.ds(j * _CHUNK, _CHUNK), :]
                rowdot = jnp.sum(Cc * d1, axis=1, keepdims=True)
                d0 = (1.0 / T) * (1.0 / (rowdot + _EPS))
                return colsum + jnp.sum(Cc * d0, axis=0, keepdims=True)

            colsum = jnp.zeros((1, E), jnp.float32)
            for j in range(n_chunks):
                colsum = chunk_body(j, colsum)
            return (1.0 / E) * (1.0 / (colsum + _EPS))

        d1 = jax.lax.fori_loop(
            jnp.int32(0), jnp.int32(_SINKHORN_ITERS), iter_body,
            jnp.ones((1, E), jnp.float32),
        )

        def argmax_body(jj):
            Sc = C_scr[pl.ds(jj, 2048), :] * d1
            m = jnp.max(Sc, axis=1, keepdims=True)
            e_iota = jax.lax.broadcasted_iota(jnp.int32, Sc.shape, 1)
            idxc = jnp.min(
                jnp.where(Sc >= m, e_iota, E), axis=1, keepdims=True
            )
            idx_ref[pl.ds(jj, 2048), :] = idxc

        for jj in range(0, T, 2048):
            argmax_body(jj)


def kernel(hidden_states, W, b):
    T, H = hidden_states.shape
    E = W.shape[1]
    grid = (T // _TILE,)

    logits, aff, idx = pl.pallas_call(
        _router_kernel,
        grid=grid,
        in_specs=[
            pl.BlockSpec((_TILE, H), lambda i: (i, _I0)),
            pl.BlockSpec((H, E), lambda i: (_I0, _I0)),
            pl.BlockSpec((1, E), lambda i: (_I0, _I0)),
        ],
        out_specs=[
            pl.BlockSpec((_TILE, E), lambda i: (i, _I0)),
            pl.BlockSpec((_TILE, E), lambda i: (i, _I0)),
            pl.BlockSpec((T, 1), lambda i: (_I0, _I0)),
        ],
        out_shape=[
            jax.ShapeDtypeStruct((T, E), jnp.float32),
            jax.ShapeDtypeStruct((T, E), jnp.float32),
            jax.ShapeDtypeStruct((T, 1), jnp.int32),
        ],
        scratch_shapes=[pltpu.VMEM((T, E), jnp.float32)],
        compiler_params=pltpu.CompilerParams(
            vmem_limit_bytes=110 * 1024 * 1024,
        ),
    )(hidden_states, W, b.reshape(1, E).astype(jnp.float32))

    return (logits, aff, idx.astype(jnp.int64))


# speedup vs baseline: 3.7018x; 3.7018x over previous
import jax
import jax.numpy as jnp
import numpy as np
from jax.experimental import pallas as pl
from jax.experimental.pallas import tpu as pltpu

_SINKHORN_ITERS = 30
_EPS = 1e-8
_TILE = 2048
_CW = 2048
_I0 = np.int32(0)


def _router_kernel(hs_ref, W_ref, b_ref, logitsT_ref, affT_ref, idx_ref, C_scr):
    i = pl.program_id(0)
    n = pl.num_programs(0)
    tile = hs_ref.shape[0]

    logitsT = (
        jax.lax.dot_general(
            W_ref[...],
            hs_ref[...],
            dimension_numbers=(((0,), (1,)), ((), ())),
            preferred_element_type=jnp.float32,
        )
        + b_ref[...]
    )
    logitsT_ref[...] = logitsT
    affT_ref[...] = jax.nn.sigmoid(logitsT)
    C_scr[:, pl.ds(i * tile, tile)] = jnp.exp(logitsT)

    @pl.when(i == n - 1)
    def _finalize():
        E, T = C_scr.shape
        n_chunks = T // _CW

        def iter_body(_, d1):
            def chunk_body(j, colsum):
                c = C_scr[:, pl.ds(j * _CW, _CW)]
                s = jnp.sum(c * d1, axis=0, keepdims=True)
                d0 = (1.0 / T) * (1.0 / (s + _EPS))
                return colsum + jnp.sum(c * d0, axis=1, keepdims=True)

            colsum = jax.lax.fori_loop(
                jnp.int32(0), jnp.int32(n_chunks), chunk_body,
                jnp.zeros((E, 1), jnp.float32),
            )
            return (1.0 / E) * (1.0 / (colsum + _EPS))

        d1 = jax.lax.fori_loop(
            jnp.int32(0), jnp.int32(_SINKHORN_ITERS), iter_body,
            jnp.ones((E, 1), jnp.float32),
        )

        def argmax_body(j, carry):
            jj = j * _CW
            S = C_scr[:, pl.ds(jj, _CW)] * d1
            m = jnp.max(S, axis=0, keepdims=True)
            e_iota = jax.lax.broadcasted_iota(jnp.int32, S.shape, 0)
            idxc = jnp.min(
                jnp.where(S >= m, e_iota, E), axis=0, keepdims=True
            )
            idx_ref[:, pl.ds(jj, _CW)] = idxc
            return carry

        jax.lax.fori_loop(
            jnp.int32(0), jnp.int32(n_chunks), argmax_body, jnp.int32(0)
        )


def kernel(hidden_states, W, b):
    T, H = hidden_states.shape
    E = W.shape[1]
    grid = (T // _TILE,)

    logitsT, affT, idx = pl.pallas_call(
        _router_kernel,
        grid=grid,
        in_specs=[
            pl.BlockSpec((_TILE, H), lambda i: (i, _I0)),
            pl.BlockSpec((H, E), lambda i: (_I0, _I0)),
            pl.BlockSpec((E, 1), lambda i: (_I0, _I0)),
        ],
        out_specs=[
            pl.BlockSpec((E, _TILE), lambda i: (_I0, i)),
            pl.BlockSpec((E, _TILE), lambda i: (_I0, i)),
            pl.BlockSpec((1, T), lambda i: (_I0, _I0)),
        ],
        out_shape=[
            jax.ShapeDtypeStruct((E, T), jnp.float32),
            jax.ShapeDtypeStruct((E, T), jnp.float32),
            jax.ShapeDtypeStruct((1, T), jnp.int32),
        ],
        scratch_shapes=[pltpu.VMEM((E, T), jnp.float32)],
        compiler_params=pltpu.CompilerParams(
            vmem_limit_bytes=100 * 1024 * 1024,
        ),
    )(hidden_states, W, b.reshape(E, 1).astype(jnp.float32))

    return (
        logitsT.T,
        affT.T,
        idx.reshape(T, 1).astype(jnp.int64),
    )


# speedup vs baseline: 5.4724x; 1.4783x over previous
import jax
import jax.numpy as jnp
import numpy as np
from jax.experimental import pallas as pl
from jax.experimental.pallas import tpu as pltpu

_SINKHORN_ITERS = 30
_EPS = 1e-8
_TILE = 4096
_CW = 1024
_I0 = np.int32(0)


def _router_kernel(hs_ref, W_ref, b_ref, logitsT_ref, affT_ref, idx_ref, C_scr):
    i = pl.program_id(0)
    n = pl.num_programs(0)
    tile = hs_ref.shape[0]

    logitsT = (
        jax.lax.dot_general(
            W_ref[...],
            hs_ref[...],
            dimension_numbers=(((0,), (1,)), ((), ())),
            preferred_element_type=jnp.float32,
        )
        + b_ref[...]
    )
    logitsT_ref[...] = logitsT
    affT_ref[...] = jax.nn.sigmoid(logitsT)
    C_scr[:, pl.ds(i * tile, tile)] = jnp.exp(logitsT)

    @pl.when(i == n - 1)
    def _finalize():
        E, T = C_scr.shape
        n_chunks = T // _CW

        def iter_body(_, d1):
            def chunk_body(j, colsum):
                c = C_scr[:, pl.ds(j * _CW, _CW)]
                s = jnp.sum(c * d1, axis=0, keepdims=True)
                d0 = (1.0 / T) * (1.0 / (s + _EPS))
                return colsum + jnp.sum(c * d0, axis=1, keepdims=True)

            colsum = jnp.zeros((E, 1), jnp.float32)
            for j in range(n_chunks):
                colsum = chunk_body(jnp.int32(j), colsum)
            return (1.0 / E) * (1.0 / (colsum + _EPS))

        d1 = jax.lax.fori_loop(
            jnp.int32(0), jnp.int32(_SINKHORN_ITERS), iter_body,
            jnp.ones((E, 1), jnp.float32),
        )

        def argmax_body(j, carry):
            jj = j * _CW
            S = C_scr[:, pl.ds(jj, _CW)] * d1
            m = jnp.max(S, axis=0, keepdims=True)
            e_iota = jax.lax.broadcasted_iota(jnp.int32, S.shape, 0)
            idxc = jnp.min(
                jnp.where(S >= m, e_iota, E), axis=0, keepdims=True
            )
            idx_ref[:, pl.ds(jj, _CW)] = idxc
            return carry

        jax.lax.fori_loop(
            jnp.int32(0), jnp.int32(n_chunks), argmax_body, jnp.int32(0)
        )


def kernel(hidden_states, W, b):
    T, H = hidden_states.shape
    E = W.shape[1]
    grid = (T // _TILE,)

    logitsT, affT, idx = pl.pallas_call(
        _router_kernel,
        grid=grid,
        in_specs=[
            pl.BlockSpec((_TILE, H), lambda i: (i, _I0)),
            pl.BlockSpec((H, E), lambda i: (_I0, _I0)),
            pl.BlockSpec((E, 1), lambda i: (_I0, _I0)),
        ],
        out_specs=[
            pl.BlockSpec((E, _TILE), lambda i: (_I0, i)),
            pl.BlockSpec((E, _TILE), lambda i: (_I0, i)),
            pl.BlockSpec((1, T), lambda i: (_I0, _I0)),
        ],
        out_shape=[
            jax.ShapeDtypeStruct((E, T), jnp.float32),
            jax.ShapeDtypeStruct((E, T), jnp.float32),
            jax.ShapeDtypeStruct((1, T), jnp.int32),
        ],
        scratch_shapes=[pltpu.VMEM((E, T), jnp.float32)],
        compiler_params=pltpu.CompilerParams(
            vmem_limit_bytes=100 * 1024 * 1024,
        ),
    )(hidden_states, W, b.reshape(E, 1).astype(jnp.float32))

    return (
        logitsT.T,
        affT.T,
        idx.reshape(T, 1).astype(jnp.int64),
    )


# speedup vs baseline: 5.8024x; 1.0603x over previous
import jax
import jax.numpy as jnp
import numpy as np
from jax.experimental import pallas as pl
from jax.experimental.pallas import tpu as pltpu

_SINKHORN_ITERS = 30
_EPS = 1e-8
_TILE = 4096
_CW = 2048
_I0 = np.int32(0)


def _router_kernel(hs_ref, W_ref, b_ref, logitsT_ref, affT_ref, idx_ref, C_scr):
    i = pl.program_id(0)
    n = pl.num_programs(0)
    tile = hs_ref.shape[0]

    logitsT = (
        jax.lax.dot_general(
            W_ref[...],
            hs_ref[...],
            dimension_numbers=(((0,), (1,)), ((), ())),
            preferred_element_type=jnp.float32,
        )
        + b_ref[...]
    )
    logitsT_ref[...] = logitsT
    affT_ref[...] = jax.nn.sigmoid(logitsT)
    C_scr[:, pl.ds(i * tile, tile)] = jnp.exp(logitsT)

    @pl.when(i == n - 1)
    def _finalize():
        E, T = C_scr.shape
        n_chunks = T // _CW

        def iter_body(_, d1):
            def chunk_body(j, colsum8):
                c = C_scr[:, pl.ds(j * _CW, _CW)]
                s = jnp.sum(c * d1, axis=0, keepdims=True)
                d0 = (1.0 / T) * (1.0 / (s + _EPS))
                d0b = jnp.broadcast_to(d0, (8, _CW))
                return colsum8 + jax.lax.dot_general(
                    c, d0b,
                    dimension_numbers=(((1,), (1,)), ((), ())),
                    preferred_element_type=jnp.float32,
                )

            colsum8 = jnp.zeros((E, 8), jnp.float32)
            for j in range(n_chunks):
                colsum8 = chunk_body(jnp.int32(j), colsum8)
            colsum = colsum8[:, :1]
            return (1.0 / E) * (1.0 / (colsum + _EPS))

        d1 = jax.lax.fori_loop(
            jnp.int32(0), jnp.int32(_SINKHORN_ITERS), iter_body,
            jnp.ones((E, 1), jnp.float32),
        )

        def argmax_body(j, carry):
            jj = j * _CW
            S = C_scr[:, pl.ds(jj, _CW)] * d1
            m = jnp.max(S, axis=0, keepdims=True)
            e_iota = jax.lax.broadcasted_iota(jnp.int32, S.shape, 0)
            idxc = jnp.min(
                jnp.where(S >= m, e_iota, E), axis=0, keepdims=True
            )
            idx_ref[:, pl.ds(jj, _CW)] = idxc
            return carry

        jax.lax.fori_loop(
            jnp.int32(0), jnp.int32(n_chunks), argmax_body, jnp.int32(0)
        )


def kernel(hidden_states, W, b):
    T, H = hidden_states.shape
    E = W.shape[1]
    grid = (T // _TILE,)

    logitsT, affT, idx = pl.pallas_call(
        _router_kernel,
        grid=grid,
        in_specs=[
            pl.BlockSpec((_TILE, H), lambda i: (i, _I0)),
            pl.BlockSpec((H, E), lambda i: (_I0, _I0)),
            pl.BlockSpec((E, 1), lambda i: (_I0, _I0)),
        ],
        out_specs=[
            pl.BlockSpec((E, _TILE), lambda i: (_I0, i)),
            pl.BlockSpec((E, _TILE), lambda i: (_I0, i)),
            pl.BlockSpec((1, T), lambda i: (_I0, _I0)),
        ],
        out_shape=[
            jax.ShapeDtypeStruct((E, T), jnp.float32),
            jax.ShapeDtypeStruct((E, T), jnp.float32),
            jax.ShapeDtypeStruct((1, T), jnp.int32),
        ],
        scratch_shapes=[pltpu.VMEM((E, T), jnp.float32)],
        compiler_params=pltpu.CompilerParams(
            vmem_limit_bytes=100 * 1024 * 1024,
        ),
    )(hidden_states, W, b.reshape(E, 1).astype(jnp.float32))

    return (
        logitsT.T,
        affT.T,
        idx.reshape(T, 1).astype(jnp.int64),
    )


# speedup vs baseline: 5.8207x; 1.0032x over previous
import jax
import jax.numpy as jnp
import numpy as np
from jax.experimental import pallas as pl
from jax.experimental.pallas import tpu as pltpu

_SINKHORN_ITERS = 30
_EPS = 1e-8
_TILE = 4096
_CW = 2048
_I0 = np.int32(0)


def _router_kernel(hs_ref, W_ref, b_ref, logitsT_ref, affT_ref, idx_ref, C_scr):
    i = pl.program_id(0)
    n = pl.num_programs(0)
    tile = hs_ref.shape[0]

    logitsT = (
        jax.lax.dot_general(
            W_ref[...],
            hs_ref[...],
            dimension_numbers=(((0,), (1,)), ((), ())),
            preferred_element_type=jnp.float32,
        )
        + b_ref[...]
    )
    logitsT_ref[...] = logitsT
    affT_ref[...] = jax.nn.sigmoid(logitsT)
    C_scr[:, pl.ds(i * tile, tile)] = jnp.exp(logitsT)

    @pl.when(i == n - 1)
    def _finalize():
        E, T = C_scr.shape
        n_chunks = T // _CW

        def iter_body(_, d1):
            def chunk_body(j, colsum8):
                c = C_scr[:, pl.ds(j * _CW, _CW)]
                s = jnp.sum(c * d1, axis=0, keepdims=True)
                d0 = (1.0 / T) * (1.0 / (s + _EPS))
                d0b = jnp.broadcast_to(d0, (8, _CW))
                return colsum8 + jax.lax.dot_general(
                    c, d0b,
                    dimension_numbers=(((1,), (1,)), ((), ())),
                    preferred_element_type=jnp.float32,
                )

            colsum8 = jnp.zeros((E, 8), jnp.float32)
            for j in range(n_chunks):
                colsum8 = chunk_body(jnp.int32(j), colsum8)
            colsum = colsum8[:, :1]
            return (1.0 / E) * (1.0 / (colsum + _EPS))

        def iter3_body(k3, d1):
            d1 = iter_body(k3, d1)
            d1 = iter_body(k3, d1)
            return iter_body(k3, d1)

        d1 = jax.lax.fori_loop(
            jnp.int32(0), jnp.int32(_SINKHORN_ITERS // 3), iter3_body,
            jnp.ones((E, 1), jnp.float32),
        )

        def argmax_body(j, carry):
            jj = j * _CW
            S = C_scr[:, pl.ds(jj, _CW)] * d1
            m = jnp.max(S, axis=0, keepdims=True)
            e_iota = jax.lax.broadcasted_iota(jnp.int32, S.shape, 0)
            idxc = jnp.min(
                jnp.where(S >= m, e_iota, E), axis=0, keepdims=True
            )
            idx_ref[:, pl.ds(jj, _CW)] = idxc
            return carry

        jax.lax.fori_loop(
            jnp.int32(0), jnp.int32(n_chunks), argmax_body, jnp.int32(0)
        )


def kernel(hidden_states, W, b):
    T, H = hidden_states.shape
    E = W.shape[1]
    grid = (T // _TILE,)

    logitsT, affT, idx = pl.pallas_call(
        _router_kernel,
        grid=grid,
        in_specs=[
            pl.BlockSpec((_TILE, H), lambda i: (i, _I0)),
            pl.BlockSpec((H, E), lambda i: (_I0, _I0)),
            pl.BlockSpec((E, 1), lambda i: (_I0, _I0)),
        ],
        out_specs=[
            pl.BlockSpec((E, _TILE), lambda i: (_I0, i)),
            pl.BlockSpec((E, _TILE), lambda i: (_I0, i)),
            pl.BlockSpec((1, T), lambda i: (_I0, _I0)),
        ],
        out_shape=[
            jax.ShapeDtypeStruct((E, T), jnp.float32),
            jax.ShapeDtypeStruct((E, T), jnp.float32),
            jax.ShapeDtypeStruct((1, T), jnp.int32),
        ],
        scratch_shapes=[pltpu.VMEM((E, T), jnp.float32)],
        compiler_params=pltpu.CompilerParams(
            vmem_limit_bytes=100 * 1024 * 1024,
        ),
    )(hidden_states, W, b.reshape(E, 1).astype(jnp.float32))

    return (
        logitsT.T,
        affT.T,
        idx.reshape(T, 1).astype(jnp.int64),
    )


# speedup vs baseline: 6.0444x; 1.0384x over previous
import jax
import jax.numpy as jnp
import numpy as np
from jax.experimental import pallas as pl
from jax.experimental.pallas import tpu as pltpu

_SINKHORN_ITERS = 30
_EPS = 1e-8
_TILE = 4096
_CW = 2048
_I0 = np.int32(0)


def _router_kernel(hs_ref, W_ref, b_ref, logitsT_ref, affT_ref, idx_ref, C_scr):
    i = pl.program_id(0)
    n = pl.num_programs(0)
    tile = hs_ref.shape[0]

    logitsT = (
        jax.lax.dot_general(
            W_ref[...],
            hs_ref[...],
            dimension_numbers=(((1,), (1,)), ((), ())),
            preferred_element_type=jnp.float32,
        )
        + jnp.swapaxes(b_ref[...], 0, 1)
    )
    logitsT_ref[...] = logitsT
    affT_ref[...] = jax.nn.sigmoid(logitsT)
    C_scr[:, pl.ds(i * tile, tile)] = jnp.exp(logitsT)

    @pl.when(i == n - 1)
    def _finalize():
        E, T = C_scr.shape
        n_chunks = T // _CW

        def iter_body(_, d1):
            def chunk_body(j, colsum8):
                c = C_scr[:, pl.ds(j * _CW, _CW)]
                s = jnp.sum(c * d1, axis=0, keepdims=True)
                d0 = (1.0 / T) * (1.0 / (s + _EPS))
                d0b = jnp.broadcast_to(d0, (8, _CW))
                return colsum8 + jax.lax.dot_general(
                    c, d0b,
                    dimension_numbers=(((1,), (1,)), ((), ())),
                    preferred_element_type=jnp.float32,
                )

            colsum8 = jnp.zeros((E, 8), jnp.float32)
            for j in range(n_chunks):
                colsum8 = chunk_body(jnp.int32(j), colsum8)
            colsum = colsum8[:, :1]
            return (1.0 / E) * (1.0 / (colsum + _EPS))

        def iter3_body(k3, d1):
            d1 = iter_body(k3, d1)
            d1 = iter_body(k3, d1)
            return iter_body(k3, d1)

        d1 = jax.lax.fori_loop(
            jnp.int32(0), jnp.int32(_SINKHORN_ITERS // 3), iter3_body,
            jnp.ones((E, 1), jnp.float32),
        )

        def argmax_body(j, carry):
            jj = j * _CW
            S = C_scr[:, pl.ds(jj, _CW)] * d1
            m = jnp.max(S, axis=0, keepdims=True)
            e_iota = jax.lax.broadcasted_iota(jnp.int32, S.shape, 0)
            idxc = jnp.min(
                jnp.where(S >= m, e_iota, E), axis=0, keepdims=True
            )
            idx_ref[:, pl.ds(jj, _CW)] = idxc
            return carry

        jax.lax.fori_loop(
            jnp.int32(0), jnp.int32(n_chunks), argmax_body, jnp.int32(0)
        )


def kernel(hidden_states, W, b):
    T, H = hidden_states.shape
    E = W.shape[1]
    grid = (T // _TILE,)

    logitsT, affT, idx = pl.pallas_call(
        _router_kernel,
        grid=grid,
        in_specs=[
            pl.BlockSpec((_TILE, H), lambda i: (i, _I0)),
            pl.BlockSpec((E, H), lambda i: (_I0, _I0)),
            pl.BlockSpec((1, E), lambda i: (_I0, _I0)),
        ],
        out_specs=[
            pl.BlockSpec((E, _TILE), lambda i: (_I0, i)),
            pl.BlockSpec((E, _TILE), lambda i: (_I0, i)),
            pl.BlockSpec((1, T), lambda i: (_I0, _I0)),
        ],
        out_shape=[
            jax.ShapeDtypeStruct((E, T), jnp.float32),
            jax.ShapeDtypeStruct((E, T), jnp.float32),
            jax.ShapeDtypeStruct((1, T), jnp.int32),
        ],
        scratch_shapes=[pltpu.VMEM((E, T), jnp.float32)],
        compiler_params=pltpu.CompilerParams(
            vmem_limit_bytes=100 * 1024 * 1024,
        ),
    )(hidden_states, W.T, b.reshape(1, E).astype(jnp.float32))

    return (
        logitsT.T,
        affT.T,
        idx.reshape(T, 1).astype(jnp.int64),
    )
